# baseline (speedup 1.0000x reference)
import jax
import jax.numpy as jnp
from jax.experimental import pallas as pl
from jax.experimental.pallas import tpu as pltpu

_CPB = 4
_SPB = 2


def _body(in_hbm, idx_ref, w2_ref, b2_ref, w3_ref, b3_ref, o_hbm,
          vbuf, obuf, t3_s, sin, sout):
    B, G, A = in_hbm.shape
    half = G // _CPB

    def in_copy(b, j):
        return pltpu.make_async_copy(
            in_hbm.at[b, pl.ds(j * half, half)],
            vbuf.at[b, pl.ds(j * half, half)],
            sin.at[b * _CPB + j])

    for b in range(B):
        for j in range(_CPB):
            in_copy(b, j).start()

    for b in range(B):
        for j in range(_CPB):
            in_copy(b, j).wait()
        s = jnp.sum(vbuf[b], axis=0, keepdims=True)
        m = jax.lax.dot_general(
            s, w2_ref[...], (((1,), (0,)), ((), ())),
            preferred_element_type=jnp.float32) + b2_ref[...]
        t3_s[pl.ds(b, 1), :] = jax.lax.dot_general(
            m, w3_ref[...], (((1,), (0,)), ((), ())),
            preferred_element_type=jnp.float32)

    iota = jax.lax.broadcasted_iota(jnp.int32, (G, B), 1)
    onehot = (idx_ref[...] == iota).astype(jnp.float32)
    msg = jax.lax.dot_general(
        onehot, t3_s[...], (((1,), (0,)), ((), ())),
        preferred_element_type=jnp.float32) + b3_ref[...]

    sh = G // _SPB

    def out_copy(b, j):
        return pltpu.make_async_copy(
            obuf.at[b, pl.ds(j * sh, sh)],
            o_hbm.at[b, pl.ds(j * sh, sh)],
            sout.at[b * _SPB + j])

    for b in range(B):
        obuf[b] = jax.lax.dot_general(
            vbuf[b], w3_ref[...], (((1,), (0,)), ((), ())),
            preferred_element_type=jnp.float32) + msg
        for j in range(_SPB):
            out_copy(b, j).start()
    for b in range(B):
        for j in range(_SPB):
            out_copy(b, j).wait()


def kernel(out, coords_neighbors_idx, n_batch, n_grid, n_ao, W2, b2, W3, b3):
    B, G, A = out.shape
    idx2d = coords_neighbors_idx.astype(jnp.int32).reshape(G, 1)
    return pl.pallas_call(
        _body,
        in_specs=[
            pl.BlockSpec(memory_space=pltpu.MemorySpace.HBM),
            pl.BlockSpec(memory_space=pltpu.MemorySpace.VMEM),
            pl.BlockSpec(memory_space=pltpu.MemorySpace.VMEM),
            pl.BlockSpec(memory_space=pltpu.MemorySpace.VMEM),
            pl.BlockSpec(memory_space=pltpu.MemorySpace.VMEM),
            pl.BlockSpec(memory_space=pltpu.MemorySpace.VMEM),
        ],
        out_specs=pl.BlockSpec(memory_space=pltpu.MemorySpace.HBM),
        out_shape=jax.ShapeDtypeStruct((B, G, A), jnp.float32),
        scratch_shapes=[
            pltpu.VMEM((B, G, A), jnp.float32),
            pltpu.VMEM((B, G, A), jnp.float32),
            pltpu.VMEM((B, A), jnp.float32),
            pltpu.SemaphoreType.DMA((B * _CPB,)),
            pltpu.SemaphoreType.DMA((B * _SPB,)),
        ],
    )(out, idx2d, W2, b2.reshape(1, A), W3, b3.reshape(1, A))


# speedup vs baseline: 1.1416x; 1.1416x over previous
import jax
import jax.numpy as jnp
from jax.experimental import pallas as pl
from jax.experimental.pallas import tpu as pltpu

_CPB = 2
_SPB = 2


def _body(in_hbm, idx_ref, w2_ref, b2_ref, w3_ref, b3_ref, o_hbm,
          vbuf, obuf, t3_s, sin, sout):
    B, G, A = in_hbm.shape
    rh = G // _CPB

    def in_copy(b, j):
        return pltpu.make_async_copy(
            in_hbm.at[b, pl.ds(j * rh, rh)],
            vbuf.at[b, pl.ds(j * rh, rh)],
            sin.at[b * _CPB + j])

    for b in range(B):
        for j in range(_CPB):
            in_copy(b, j).start()

    b2r = jnp.reshape(b2_ref[...], (1, A))
    b3r = jnp.reshape(b3_ref[...], (1, A))

    for b in range(B):
        for j in range(_CPB):
            in_copy(b, j).wait()
        s = jnp.sum(vbuf[b], axis=0, keepdims=True)
        m = jax.lax.dot_general(
            s, w2_ref[...], (((1,), (0,)), ((), ())),
            preferred_element_type=jnp.float32) + b2r
        t3_s[pl.ds(b, 1), :] = jax.lax.dot_general(
            m, w3_ref[...], (((1,), (0,)), ((), ())),
            preferred_element_type=jnp.float32) + b3r

    idxb = jax.lax.broadcast_in_dim(idx_ref[...], (B, G), (1,))
    iota = jax.lax.broadcasted_iota(jnp.int32, (B, G), 0)
    onehot_t = (idxb == iota).astype(jnp.float32)
    msg = jax.lax.dot_general(
        onehot_t, t3_s[...], (((0,), (0,)), ((), ())),
        preferred_element_type=jnp.float32)

    sh = G // _SPB

    def out_copy(b, j):
        return pltpu.make_async_copy(
            obuf.at[b, pl.ds(j * sh, sh)],
            o_hbm.at[b, pl.ds(j * sh, sh)],
            sout.at[b * _SPB + j])

    for b in range(B):
        obuf[b] = jax.lax.dot_general(
            vbuf[b], w3_ref[...], (((1,), (0,)), ((), ())),
            preferred_element_type=jnp.float32) + msg
        for j in range(_SPB):
            out_copy(b, j).start()
    for b in range(B):
        for j in range(_SPB):
            out_copy(b, j).wait()


def kernel(out, coords_neighbors_idx, n_batch, n_grid, n_ao, W2, b2, W3, b3):
    B, G, A = out.shape
    idx32 = coords_neighbors_idx.astype(jnp.int32)
    return pl.pallas_call(
        _body,
        in_specs=[
            pl.BlockSpec(memory_space=pltpu.MemorySpace.HBM),
            pl.BlockSpec(memory_space=pltpu.MemorySpace.VMEM),
            pl.BlockSpec(memory_space=pltpu.MemorySpace.VMEM),
            pl.BlockSpec(memory_space=pltpu.MemorySpace.VMEM),
            pl.BlockSpec(memory_space=pltpu.MemorySpace.VMEM),
            pl.BlockSpec(memory_space=pltpu.MemorySpace.VMEM),
        ],
        out_specs=pl.BlockSpec(memory_space=pltpu.MemorySpace.HBM),
        out_shape=jax.ShapeDtypeStruct((B, G, A), jnp.float32),
        scratch_shapes=[
            pltpu.VMEM((B, G, A), jnp.float32),
            pltpu.VMEM((B, G, A), jnp.float32),
            pltpu.VMEM((B, A), jnp.float32),
            pltpu.SemaphoreType.DMA((B * _CPB,)),
            pltpu.SemaphoreType.DMA((B * _SPB,)),
        ],
    )(out, idx32, W2, b2, W3, b3)


# speedup vs baseline: 1.1422x; 1.0005x over previous
import jax
import jax.numpy as jnp
from jax.experimental import pallas as pl
from jax.experimental.pallas import tpu as pltpu

_CPB = 2
_SPB = 4


def _body(in_hbm, idx_ref, w2_ref, b2_ref, w3_ref, b3_ref, o_hbm,
          vbuf, obuf, t3_s, sin, sout):
    B, G, A = in_hbm.shape
    rh = G // _CPB

    def in_copy(b, j):
        return pltpu.make_async_copy(
            in_hbm.at[b, pl.ds(j * rh, rh)],
            vbuf.at[b, pl.ds(j * rh, rh)],
            sin.at[b * _CPB + j])

    for b in range(B):
        for j in range(_CPB):
            in_copy(b, j).start()

    b2r = jnp.reshape(b2_ref[...], (1, A))
    b3r = jnp.reshape(b3_ref[...], (1, A))

    idxb = jax.lax.broadcast_in_dim(idx_ref[...], (B, G), (1,))
    iota = jax.lax.broadcasted_iota(jnp.int32, (B, G), 0)
    onehot_t = (idxb == iota).astype(jnp.float32)

    for b in range(B):
        for j in range(_CPB):
            in_copy(b, j).wait()
        s = jnp.sum(vbuf[b], axis=0, keepdims=True)
        m = jax.lax.dot_general(
            s, w2_ref[...], (((1,), (0,)), ((), ())),
            preferred_element_type=jnp.float32) + b2r
        t3_s[pl.ds(b, 1), :] = jax.lax.dot_general(
            m, w3_ref[...], (((1,), (0,)), ((), ())),
            preferred_element_type=jnp.float32) + b3r

    msg = jax.lax.dot_general(
        onehot_t, t3_s[...], (((0,), (0,)), ((), ())),
        preferred_element_type=jnp.float32)

    sh = G // _SPB

    def out_copy(b, j):
        return pltpu.make_async_copy(
            obuf.at[b, pl.ds(j * sh, sh)],
            o_hbm.at[b, pl.ds(j * sh, sh)],
            sout.at[b * _SPB + j])

    for b in range(B):
        for j in range(_SPB):
            sl = pl.ds(j * sh, sh)
            obuf[b, sl] = jax.lax.dot_general(
                vbuf[b, sl], w3_ref[...], (((1,), (0,)), ((), ())),
                preferred_element_type=jnp.float32) + msg[j * sh:(j + 1) * sh]
            out_copy(b, j).start()
    for b in range(B):
        for j in range(_SPB):
            out_copy(b, j).wait()


def kernel(out, coords_neighbors_idx, n_batch, n_grid, n_ao, W2, b2, W3, b3):
    B, G, A = out.shape
    idx32 = coords_neighbors_idx.astype(jnp.int32)
    return pl.pallas_call(
        _body,
        in_specs=[
            pl.BlockSpec(memory_space=pltpu.MemorySpace.HBM),
            pl.BlockSpec(memory_space=pltpu.MemorySpace.VMEM),
            pl.BlockSpec(memory_space=pltpu.MemorySpace.VMEM),
            pl.BlockSpec(memory_space=pltpu.MemorySpace.VMEM),
            pl.BlockSpec(memory_space=pltpu.MemorySpace.VMEM),
            pl.BlockSpec(memory_space=pltpu.MemorySpace.VMEM),
        ],
        out_specs=pl.BlockSpec(memory_space=pltpu.MemorySpace.HBM),
        out_shape=jax.ShapeDtypeStruct((B, G, A), jnp.float32),
        scratch_shapes=[
            pltpu.VMEM((B, G, A), jnp.float32),
            pltpu.VMEM((B, G, A), jnp.float32),
            pltpu.VMEM((B, A), jnp.float32),
            pltpu.SemaphoreType.DMA((B * _CPB,)),
            pltpu.SemaphoreType.DMA((B * _SPB,)),
        ],
    )(out, idx32, W2, b2, W3, b3)
